# baseline (device time: 45163 ns/iter reference)
import jax
import jax.numpy as jnp
from jax import lax
from jax.experimental import pallas as pl
from jax.experimental.pallas import tpu as pltpu

TC = 32


def kernel(x, A, B, C):
    b, s_loc, d = x.shape
    n = A.shape[1]
    n_local = s_loc // TC
    n_q = n_local // 2

    my_x = lax.axis_index("x")
    my_y = lax.axis_index("y")
    sinfo = jnp.stack(
        [my_y * n_q,
         jnp.where((my_x == 0) & (my_y == 0), n_local - 1, n_q - 1)]
    ).astype(jnp.int32)

    def body(sref, x_ref, a_ref, b_ref, c_ref, xt_ref, bt_ref, out_ref,
             sbuf, rbuf, carry, ybuf,
             h_send_sem, h_recv_sem, o_send_sems, o_recv_sems):
        i = pl.program_id(0)
        mx = lax.axis_index("x")
        my = lax.axis_index("y")
        xpartner = (1 - mx, my)
        ypartner = (mx, 1 - my)

        dAt = jnp.exp(a_ref[...]).T

        @pl.when(i == 0)
        def _():
            barrier_sem = pltpu.get_barrier_semaphore()
            for nbr in (xpartner, ypartner):
                pl.semaphore_signal(
                    barrier_sem, inc=1,
                    device_id=nbr, device_id_type=pl.DeviceIdType.MESH,
                )
            pl.semaphore_wait(barrier_sem, 2)

            h = jnp.zeros((b, n, d), jnp.float32)
            for s in range(TC):
                xs = xt_ref[:, s, :]
                bs = bt_ref[:, s, :]
                h = h * dAt[None] + xs[:, None, :] * bs[:, :, None]
            sbuf[...] = h

            copy = pltpu.make_async_remote_copy(
                src_ref=sbuf, dst_ref=rbuf,
                send_sem=h_send_sem, recv_sem=h_recv_sem,
                device_id=xpartner, device_id_type=pl.DeviceIdType.MESH,
            )

            @pl.when((mx == 0) & (my == 0))
            def _():
                copy.start()
                copy.wait_send()

            @pl.when((mx == 1) & (my == 0))
            def _():
                copy.wait_recv()

            carry[...] = jnp.where(
                my == 1, sbuf[...], jnp.where(mx == 1, rbuf[...], 0.0))

        h = carry[...]
        for s in range(TC):
            xs = x_ref[:, s, :]
            bs = b_ref[:, s, :]
            cs = c_ref[:, s, :]
            h = h * dAt[None] + xs[:, None, :] * bs[:, :, None]
            ybuf[:, s, :] = jnp.sum(h * cs[:, :, None], axis=1)
        carry[...] = h

        out_ref[:, my, i, :, :] = ybuf[...].astype(jnp.bfloat16)

        send = pltpu.make_async_remote_copy(
            src_ref=out_ref.at[:, my, i],
            dst_ref=out_ref.at[:, my, i],
            send_sem=o_send_sems.at[i],
            recv_sem=o_recv_sems.at[i],
            device_id=ypartner, device_id_type=pl.DeviceIdType.MESH,
        )
        send.start()

        @pl.when(i == n_q - 1)
        def _():
            for j in range(n_q):
                drain = pltpu.make_async_remote_copy(
                    src_ref=out_ref.at[:, my, j],
                    dst_ref=out_ref.at[:, 1 - my, j],
                    send_sem=o_send_sems.at[j],
                    recv_sem=o_recv_sems.at[j],
                    device_id=ypartner, device_id_type=pl.DeviceIdType.MESH,
                )
                drain.wait_send()
                drain.wait_recv()

            def exit_barrier(sem):
                for nbr in (xpartner, ypartner):
                    pl.semaphore_signal(
                        sem, inc=1,
                        device_id=nbr, device_id_type=pl.DeviceIdType.MESH,
                    )
                pl.semaphore_wait(sem, 2)
            pl.run_scoped(exit_barrier, pltpu.SemaphoreType.REGULAR)

    y5 = pl.pallas_call(
        body,
        grid_spec=pltpu.PrefetchScalarGridSpec(
            num_scalar_prefetch=1,
            grid=(n_q,),
            in_specs=[
                pl.BlockSpec((b, TC, d), lambda i, s: (0, s[0] + i, 0)),
                pl.BlockSpec(memory_space=pltpu.VMEM),
                pl.BlockSpec((b, TC, n), lambda i, s: (0, s[0] + i, 0)),
                pl.BlockSpec((b, TC, n), lambda i, s: (0, s[0] + i, 0)),
                pl.BlockSpec((b, TC, d), lambda i, s: (0, s[1], 0)),
                pl.BlockSpec((b, TC, n), lambda i, s: (0, s[1], 0)),
            ],
            out_specs=pl.BlockSpec(memory_space=pltpu.VMEM),
            scratch_shapes=[
                pltpu.VMEM((b, n, d), jnp.float32),
                pltpu.VMEM((b, n, d), jnp.float32),
                pltpu.VMEM((b, n, d), jnp.float32),
                pltpu.VMEM((b, TC, d), jnp.float32),
                pltpu.SemaphoreType.DMA,
                pltpu.SemaphoreType.DMA,
                pltpu.SemaphoreType.DMA((n_q,)),
                pltpu.SemaphoreType.DMA((n_q,)),
            ],
        ),
        out_shape=jax.ShapeDtypeStruct((b, 2, n_q, TC, d), jnp.bfloat16),
        compiler_params=pltpu.CompilerParams(
            collective_id=0,
            dimension_semantics=("arbitrary",),
        ),
    )(sinfo, x, A, B, C, x, B)

    return y5.reshape(b, s_loc, d)


# device time: 41354 ns/iter; 1.0921x vs baseline; 1.0921x over previous
import jax
import jax.numpy as jnp
from jax import lax
from jax.experimental import pallas as pl
from jax.experimental.pallas import tpu as pltpu

TC = 32


def kernel(x, A, B, C):
    b, s_loc, d = x.shape
    n = A.shape[1]
    n_local = s_loc // TC
    n_q = n_local // 2

    my_x = lax.axis_index("x")
    my_y = lax.axis_index("y")
    sinfo = jnp.stack(
        [my_y * n_q,
         jnp.where((my_x == 0) & (my_y == 0), n_local - 1, n_q - 1)]
    ).astype(jnp.int32)

    def body(sref, x_ref, a_ref, b_ref, c_ref, xt_ref, bt_ref, out_ref,
             sbuf, rbuf, carry, ybuf,
             h_send_sem, h_recv_sem, o_send_sems, o_recv_sems):
        i = pl.program_id(0)
        mx = lax.axis_index("x")
        my = lax.axis_index("y")
        xpartner = (1 - mx, my)
        ypartner = (mx, 1 - my)

        dAt = jnp.exp(a_ref[...]).T

        @pl.when(i == 0)
        def _():
            barrier_sem = pltpu.get_barrier_semaphore()
            for nbr in (xpartner, ypartner):
                pl.semaphore_signal(
                    barrier_sem, inc=1,
                    device_id=nbr, device_id_type=pl.DeviceIdType.MESH,
                )
            pl.semaphore_wait(barrier_sem, 2)

            h = jnp.zeros((b, n, d), jnp.float32)
            for s in range(TC):
                xs = xt_ref[:, s, :]
                bs = bt_ref[:, s, :]
                h = h * dAt[None] + xs[:, None, :] * bs[:, :, None]
            sbuf[...] = h

            copy = pltpu.make_async_remote_copy(
                src_ref=sbuf, dst_ref=rbuf,
                send_sem=h_send_sem, recv_sem=h_recv_sem,
                device_id=xpartner, device_id_type=pl.DeviceIdType.MESH,
            )

            @pl.when((mx == 0) & (my == 0))
            def _():
                copy.start()
                copy.wait_send()

            @pl.when((mx == 1) & (my == 0))
            def _():
                copy.wait_recv()

            carry[...] = jnp.where(
                my == 1, sbuf[...], jnp.where(mx == 1, rbuf[...], 0.0)
            ).astype(jnp.bfloat16)

        bd_mask = (
            lax.broadcasted_iota(jnp.int32, (b, b * n), 1) // n
            == lax.broadcasted_iota(jnp.int32, (b, b * n), 0)
        ).astype(jnp.bfloat16)
        dAb = dAt.astype(jnp.bfloat16)
        x_blk = x_ref[...].astype(jnp.bfloat16)
        b_blk = b_ref[...].astype(jnp.bfloat16)
        c_blk = c_ref[...].astype(jnp.bfloat16)
        h = carry[...]
        for s in range(TC):
            xs = x_blk[:, s, :]
            bs = b_blk[:, s, :]
            cs = c_blk[:, s, :]
            h = h * dAb[None] + xs[:, None, :] * bs[:, :, None]
            cmat = jnp.concatenate([cs] * b, axis=1) * bd_mask
            ybuf[:, s, :] = jnp.dot(
                cmat, h.reshape(b * n, d),
                preferred_element_type=jnp.float32)
        carry[...] = h

        out_ref[:, my, i, :, :] = ybuf[...].astype(jnp.bfloat16)

        send = pltpu.make_async_remote_copy(
            src_ref=out_ref.at[:, my, i],
            dst_ref=out_ref.at[:, my, i],
            send_sem=o_send_sems.at[i],
            recv_sem=o_recv_sems.at[i],
            device_id=ypartner, device_id_type=pl.DeviceIdType.MESH,
        )
        send.start()

        @pl.when(i == n_q - 1)
        def _():
            for j in range(n_q):
                drain = pltpu.make_async_remote_copy(
                    src_ref=out_ref.at[:, my, j],
                    dst_ref=out_ref.at[:, 1 - my, j],
                    send_sem=o_send_sems.at[j],
                    recv_sem=o_recv_sems.at[j],
                    device_id=ypartner, device_id_type=pl.DeviceIdType.MESH,
                )
                drain.wait_send()
                drain.wait_recv()

            def exit_barrier(sem):
                for nbr in (xpartner, ypartner):
                    pl.semaphore_signal(
                        sem, inc=1,
                        device_id=nbr, device_id_type=pl.DeviceIdType.MESH,
                    )
                pl.semaphore_wait(sem, 2)
            pl.run_scoped(exit_barrier, pltpu.SemaphoreType.REGULAR)

    y5 = pl.pallas_call(
        body,
        grid_spec=pltpu.PrefetchScalarGridSpec(
            num_scalar_prefetch=1,
            grid=(n_q,),
            in_specs=[
                pl.BlockSpec((b, TC, d), lambda i, s: (0, s[0] + i, 0)),
                pl.BlockSpec(memory_space=pltpu.VMEM),
                pl.BlockSpec((b, TC, n), lambda i, s: (0, s[0] + i, 0)),
                pl.BlockSpec((b, TC, n), lambda i, s: (0, s[0] + i, 0)),
                pl.BlockSpec((b, TC, d), lambda i, s: (0, s[1], 0)),
                pl.BlockSpec((b, TC, n), lambda i, s: (0, s[1], 0)),
            ],
            out_specs=pl.BlockSpec(memory_space=pltpu.VMEM),
            scratch_shapes=[
                pltpu.VMEM((b, n, d), jnp.float32),
                pltpu.VMEM((b, n, d), jnp.float32),
                pltpu.VMEM((b, n, d), jnp.bfloat16),
                pltpu.VMEM((b, TC, d), jnp.float32),
                pltpu.SemaphoreType.DMA,
                pltpu.SemaphoreType.DMA,
                pltpu.SemaphoreType.DMA((n_q,)),
                pltpu.SemaphoreType.DMA((n_q,)),
            ],
        ),
        out_shape=jax.ShapeDtypeStruct((b, 2, n_q, TC, d), jnp.bfloat16),
        compiler_params=pltpu.CompilerParams(
            collective_id=0,
            dimension_semantics=("arbitrary",),
        ),
    )(sinfo, x, A, B, C, x, B)

    return y5.reshape(b, s_loc, d)
